# baseline (speedup 1.0000x reference)
import functools

import jax
import jax.numpy as jnp
from jax import lax
from jax.experimental import pallas as pl
from jax.experimental.pallas import tpu as pltpu
from jax.experimental.pallas import tpu_sc as plsc

NC = 2
NS = 16
NW = NC * NS
LANES = 16
CHUNK_ROWS = 16
CHUNK = CHUNK_ROWS * 128
NBUF = 2
SB = 8


def _sc_edge_kernel(V, E, BM):
  n_chunks = E // CHUNK
  assert n_chunks * CHUNK == E
  base_t, extra = divmod(n_chunks, NW)
  assert base_t >= NBUF
  max_n = base_t + (1 if extra else 0)
  n_groups = (max_n + NBUF - 1) // NBUF
  bm_slice = BM // NS
  assert bm_slice % 128 == 0
  mesh = plsc.VectorSubcoreMesh(core_axis_name="c", subcore_axis_name="s")

  def body(vp_hbm, gm_hbm, ef_hbm, out_hbm,
           vp_v, src_v, dst_v, ef_v, bm_v, mrg_v, or_v, bm_sh, in_sems):
    cid = lax.axis_index("c")
    sid = lax.axis_index("s")
    wid = sid * NC + cid

    pltpu.sync_copy(vp_hbm, vp_v)

    def zero_body(i, _):
      bm_v[pl.ds(i * LANES, LANES)] = jnp.zeros((LANES,), jnp.int32)
      return ()
    lax.fori_loop(0, BM // LANES, zero_body, ())

    n_mine = base_t + jnp.where(wid < extra, 1, 0)

    def start_inputs(t, b):
      c = wid + t * NW
      pltpu.async_copy(gm_hbm.at[0, pl.ds(c * CHUNK, CHUNK)], src_v[b],
                       in_sems[b])
      pltpu.async_copy(gm_hbm.at[1, pl.ds(c * CHUNK, CHUNK)], dst_v[b],
                       in_sems[b])
      pltpu.async_copy(ef_hbm.at[pl.ds(c * CHUNK, CHUNK)], ef_v[b],
                       in_sems[b])

    def wait_inputs(b):
      pltpu.make_async_copy(gm_hbm.at[0, pl.ds(0, CHUNK)], src_v[b],
                            in_sems[b]).wait()
      pltpu.make_async_copy(gm_hbm.at[1, pl.ds(0, CHUNK)], dst_v[b],
                            in_sems[b]).wait()
      pltpu.make_async_copy(ef_hbm.at[pl.ds(0, CHUNK)], ef_v[b],
                            in_sems[b]).wait()

    start_inputs(0, 0)

    def group_body(g, _):
      for b in range(NBUF):
        t0 = g * NBUF + b

        @pl.when(t0 < n_mine)
        def _(t0=t0, b=b):
          wait_inputs(b)

          @pl.when(t0 + 1 < n_mine)
          def _():
            start_inputs(t0 + 1, 1 - b)

          def sb_body(sb, _):
            ws, bits, ms = [], [], []
            for j in range(SB):
              k = sb * SB + j
              s16 = src_v[b][pl.ds(k * LANES, LANES)]
              e16 = ef_v[b][pl.ds(k * LANES, LANES)]
              v16 = plsc.load_gather(vp_v, [s16])
              t16 = e16 * v16 + (1.0 - e16) * 0.5
              m = t16 > 0.5
              d16 = dst_v[b][pl.ds(k * LANES, LANES)]
              w16 = lax.shift_right_logical(d16, 5)
              b16 = lax.shift_left(jnp.int32(1), d16 & 31)
              cur = plsc.load_gather(bm_v, [w16])
              plsc.store_scatter(bm_v, [w16], cur | b16, mask=m)
              ws.append(w16)
              bits.append(b16)
              ms.append(m)

            miss0 = ms[0]
            for j in range(1, SB):
              miss0 = miss0 | ms[j]

            def retry_cond(miss):
              return plsc.all_reduce_population_count(miss)[0] > 0

            def retry_body(miss):
              nm = None
              for j in range(SB):
                cur = plsc.load_gather(bm_v, [ws[j]])
                mj = ms[j] & ((cur & bits[j]) == 0)
                plsc.store_scatter(bm_v, [ws[j]], cur | bits[j], mask=mj)
                nm = mj if nm is None else (nm | mj)
              return nm
            lax.while_loop(retry_cond, retry_body, miss0)
            return ()
          lax.fori_loop(0, CHUNK // (LANES * SB), sb_body, ())
      return ()
    lax.fori_loop(0, n_groups, group_body, ())

    pltpu.sync_copy(bm_v, bm_sh.at[pl.ds(sid * BM, BM)])
    plsc.subcore_barrier()
    for r in range(NS):
      pltpu.sync_copy(bm_sh.at[pl.ds(r * BM + sid * bm_slice, bm_slice)],
                      mrg_v.at[pl.ds(r * bm_slice, bm_slice)])

    def or_body(i, _):
      acc = mrg_v[pl.ds(i * LANES, LANES)]
      for r in range(1, NS):
        acc = acc | mrg_v[pl.ds(r * bm_slice + i * LANES, LANES)]
      or_v[pl.ds(i * LANES, LANES)] = acc
      return ()
    lax.fori_loop(0, bm_slice // LANES, or_body, ())
    pltpu.sync_copy(or_v, out_hbm.at[cid, pl.ds(sid * bm_slice, bm_slice)])

  return pl.kernel(
      body,
      out_type=jax.ShapeDtypeStruct((NC, BM), jnp.int32),
      mesh=mesh,
      compiler_params=pltpu.CompilerParams(needs_layout_passes=False),
      scratch_types=[
          pltpu.VMEM((V,), jnp.float32),
          [pltpu.VMEM((CHUNK,), jnp.int32) for _ in range(NBUF)],
          [pltpu.VMEM((CHUNK,), jnp.int32) for _ in range(NBUF)],
          [pltpu.VMEM((CHUNK,), jnp.float32) for _ in range(NBUF)],
          pltpu.VMEM((BM,), jnp.int32),
          pltpu.VMEM((BM,), jnp.int32),
          pltpu.VMEM((BM // NS,), jnp.int32),
          pltpu.VMEM_SHARED((NS * BM,), jnp.int32),
          [pltpu.SemaphoreType.DMA for _ in range(NBUF)],
      ],
  )


def _tc_epilogue_kernel(F_pad, B):
  rows = F_pad // 128
  block_rows = next(b for b in (128, 112, 96, 80, 64, 56, 48, 40, 32, 24, 16, 8)
                    if rows % b == 0)
  grid = rows // block_rows

  def body(bits_ref, bfm_ref, cv_ref, sat_ref, unsat_ref, acc_bv, acc_ms):
    g = pl.program_id(0)

    @pl.when(g == 0)
    def _():
      acc_bv[...] = jnp.zeros((B, 128), jnp.float32)
      acc_ms[...] = jnp.zeros((B, 128), jnp.float32)

    w = bits_ref[0] | bits_ref[1]
    col = lax.broadcasted_iota(jnp.int32, (block_rows, 128), 1)
    wsel = jnp.where(
        col < 32, w[:, 0:1],
        jnp.where(col < 64, w[:, 1:2],
                  jnp.where(col < 96, w[:, 2:3], w[:, 3:4])))
    cv = (lax.shift_right_logical(wsel, col & 31) & 1).astype(jnp.float32)
    cv_ref[...] = cv
    b = bfm_ref[...]
    for k in range(B):
      m = b == k
      acc_bv[k:k + 1, :] += jnp.sum(jnp.where(m, cv, 0.0), axis=0,
                                    keepdims=True)
      acc_ms[k:k + 1, :] += jnp.sum(m.astype(jnp.float32), axis=0,
                                    keepdims=True)

    @pl.when(g == grid - 1)
    def _():
      bv = jnp.sum(acc_bv[...], axis=1, keepdims=True)
      ms = jnp.sum(acc_ms[...], axis=1, keepdims=True)
      sat_ref[...] = jnp.broadcast_to(
          (ms == bv).astype(jnp.float32), (B, 128))
      unsat_ref[...] = jnp.broadcast_to(ms - bv, (B, 128))

  return pl.pallas_call(
      body,
      grid=(grid,),
      in_specs=[
          pl.BlockSpec((2, block_rows, 4), lambda g: (0, g, 0)),
          pl.BlockSpec((block_rows, 128), lambda g: (g, 0)),
      ],
      out_specs=[
          pl.BlockSpec((block_rows, 128), lambda g: (g, 0)),
          pl.BlockSpec((B, 128), lambda g: (0, 0)),
          pl.BlockSpec((B, 128), lambda g: (0, 0)),
      ],
      out_shape=[
          jax.ShapeDtypeStruct((rows, 128), jnp.float32),
          jax.ShapeDtypeStruct((B, 128), jnp.float32),
          jax.ShapeDtypeStruct((B, 128), jnp.float32),
      ],
      scratch_shapes=[
          pltpu.VMEM((B, 128), jnp.float32),
          pltpu.VMEM((B, 128), jnp.float32),
      ],
  )


@jax.jit
def kernel(variable_prediction, graph_map, batch_variable_map,
           batch_function_map, edge_feature):
  V = variable_prediction.shape[0]
  E = graph_map.shape[1]
  F = batch_function_map.shape[0]
  B = 16
  F_pad = ((F + 2047) // 2048) * 2048
  BM = ((F_pad // 32 + NS * 128 - 1) // (NS * 128)) * (NS * 128)

  V_pad = ((V + 127) // 128) * 128
  vp = jnp.concatenate(
      [variable_prediction.reshape(V),
       jnp.zeros((V_pad - V,), jnp.float32)])
  ef1 = edge_feature.reshape(E)

  bits = _sc_edge_kernel(V_pad, E, BM)(vp, graph_map, ef1)

  bfm_pad = jnp.concatenate(
      [batch_function_map,
       jnp.full((F_pad - F,), B, jnp.int32)]).reshape(F_pad // 128, 128)
  bits3 = bits[:, :F_pad // 32].reshape(2, F_pad // 128, 4)

  cv, sat, unsat = _tc_epilogue_kernel(F_pad, B)(bits3, bfm_pad)

  clause_values = cv.reshape(F_pad)[:F][:, None]
  sat_flag = sat[:, :1]
  unsat_count = unsat[:, :1]
  return (sat_flag, unsat_count, clause_values)


# speedup vs baseline: 2.5116x; 2.5116x over previous
import functools

import jax
import jax.numpy as jnp
from jax import lax
from jax.experimental import pallas as pl
from jax.experimental.pallas import tpu as pltpu
from jax.experimental.pallas import tpu_sc as plsc

NC = 2
NS = 16
NW = NC * NS
LANES = 16
CHUNK_ROWS = 16
CHUNK = CHUNK_ROWS * 128


NBUF = 2
STAGE = 2 * CHUNK
SB = 16


def _sc_edge_kernel(V, V_pad, E, F, F_pad):
  n_chunks = E // CHUNK
  assert n_chunks * CHUNK == E
  base_t, extra = divmod(n_chunks, NW)
  assert base_t >= NBUF
  max_n = base_t + (1 if extra else 0)
  n_groups = (max_n + NBUF - 1) // NBUF
  acc_slice = F_pad // NS
  mesh = plsc.VectorSubcoreMesh(core_axis_name="c", subcore_axis_name="s")

  def body(vp_hbm, gm_hbm, ef_hbm, out_hbm,
           vp_v, src_v, dst_v, ef_v, ones_v, stage_v, fire_v, cnt_v, acc_sh,
           in_sems, sc_sems):
    cid = lax.axis_index("c")
    sid = lax.axis_index("s")
    wid = sid * NC + cid

    pltpu.sync_copy(vp_hbm, vp_v.at[pl.ds(0, V)])

    def zero_body(i, _):
      ones_v[pl.ds(i * LANES, LANES)] = jnp.zeros((LANES,), jnp.float32)
      return ()
    lax.fori_loop(0, CHUNK // LANES, zero_body, ())
    base = sid * acc_slice
    n_full, rem = divmod(acc_slice, CHUNK)
    for i in range(n_full):
      pltpu.sync_copy(ones_v, acc_sh.at[pl.ds(base + i * CHUNK, CHUNK)])
    if rem:
      pltpu.sync_copy(ones_v.at[pl.ds(0, rem)],
                      acc_sh.at[pl.ds(base + n_full * CHUNK, rem)])

    def ones_body(i, _):
      ones_v[pl.ds(i * LANES, LANES)] = jnp.ones((LANES,), jnp.float32)
      return ()
    lax.fori_loop(0, CHUNK // LANES, ones_body, ())

    trash16 = F + lax.iota(jnp.int32, LANES)

    def trash_body(i, _):
      stage_v[pl.ds(i * LANES, LANES)] = trash16
      return ()
    lax.fori_loop(0, (STAGE + LANES) // LANES, trash_body, ())
    plsc.subcore_barrier()

    n_mine = base_t + jnp.where(wid < extra, 1, 0)

    def start_inputs(t, b):
      c = wid + t * NW
      pltpu.async_copy(gm_hbm.at[0, pl.ds(c * CHUNK, CHUNK)], src_v[b],
                       in_sems[b])
      pltpu.async_copy(gm_hbm.at[1, pl.ds(c * CHUNK, CHUNK)], dst_v[b],
                       in_sems[b])
      pltpu.async_copy(ef_hbm.at[pl.ds(c * CHUNK, CHUNK)], ef_v[b],
                       in_sems[b])

    def wait_inputs(b):
      pltpu.make_async_copy(gm_hbm.at[0, pl.ds(0, CHUNK)], src_v[b],
                            in_sems[b]).wait()
      pltpu.make_async_copy(gm_hbm.at[1, pl.ds(0, CHUNK)], dst_v[b],
                            in_sems[b]).wait()
      pltpu.make_async_copy(ef_hbm.at[pl.ds(0, CHUNK)], ef_v[b],
                            in_sems[b]).wait()

    def wait_scatter(h):
      pltpu.make_async_copy(ones_v, acc_sh.at[fire_v[h]], sc_sems[h]).wait()

    def fire_half(h):
      def cp_body(i, _):
        fire_v[h][pl.ds(i * LANES, LANES)] = (
            stage_v[pl.ds(h * CHUNK + i * LANES, LANES)])
        return ()
      lax.fori_loop(0, CHUNK // LANES, cp_body, ())
      pltpu.async_copy(ones_v, acc_sh.at[fire_v[h]], sc_sems[h], add=True)

    start_inputs(0, 0)

    def group_body(g, carry):
      for b in range(NBUF):
        t0 = g * NBUF + b
        ptr_s, next_fire, d, fcnt = carry

        def run_chunk(args):
          ptr_s, next_fire, d, fcnt = args
          wait_inputs(b)

          @pl.when(t0 + 1 < n_mine)
          def _():
            start_inputs(t0 + 1, 1 - b)

          def sb_body(sb, ptr_s):
            ms, ds16, cnts = [], [], []
            for j in range(SB):
              k = sb * SB + j
              s16 = src_v[b][pl.ds(k * LANES, LANES)]
              e16 = ef_v[b][pl.ds(k * LANES, LANES)]
              v16 = plsc.load_gather(vp_v, [s16])
              t16 = e16 * v16 + (1.0 - e16) * 0.5
              m = t16 > 0.5
              cnts.append(plsc.all_reduce_population_count(m)[0])
              ms.append(m)
              ds16.append(dst_v[b][pl.ds(k * LANES, LANES)])
            base = ptr_s
            for j in range(SB):
              off = base & (STAGE - 1)
              plsc.store_compressed(stage_v.at[pl.ds(off, LANES)], ds16[j],
                                    mask=ms[j])

              @pl.when(off >= STAGE - (LANES - 1))
              def _():
                stage_v[pl.ds(0, LANES)] = stage_v[pl.ds(STAGE, LANES)]
              base = base + cnts[j]
            return base
          ptr_s = lax.fori_loop(0, CHUNK // (LANES * SB), sb_body, ptr_s)

          fired = ptr_s >= next_fire
          for h in range(2):
            @pl.when(jnp.logical_and(fired, d == h))
            def _(h=h):
              @pl.when(fcnt >= 2)
              def _():
                wait_scatter(h)
              fire_half(h)
          return (ptr_s,
                  jnp.where(fired, next_fire + CHUNK, next_fire),
                  jnp.where(fired, 1 - d, d),
                  fcnt + fired.astype(jnp.int32))

        carry = lax.cond(t0 < n_mine, run_chunk, lambda a: a,
                         (ptr_s, next_fire, d, fcnt))
      return carry

    carry0 = (jnp.int32(0), jnp.int32(CHUNK),
              jnp.int32(0), jnp.int32(0))
    _, _, _, fcnt = lax.fori_loop(0, n_groups, group_body, carry0)

    for h in range(2):
      @pl.when(fcnt >= h + 1)
      def _(h=h):
        wait_scatter(h)
    for h in range(2):
      fire_half(h)
    for h in range(2):
      wait_scatter(h)

    plsc.subcore_barrier()
    pltpu.sync_copy(acc_sh.at[pl.ds(sid * acc_slice, acc_slice)],
                    out_hbm.at[cid, pl.ds(sid * acc_slice, acc_slice)])

  return pl.kernel(
      body,
      out_type=jax.ShapeDtypeStruct((NC, F_pad), jnp.float32),
      mesh=mesh,
      compiler_params=pltpu.CompilerParams(needs_layout_passes=False),
      scratch_types=[
          pltpu.VMEM((V_pad,), jnp.float32),
          [pltpu.VMEM((CHUNK,), jnp.int32) for _ in range(NBUF)],
          [pltpu.VMEM((CHUNK,), jnp.int32) for _ in range(NBUF)],
          [pltpu.VMEM((CHUNK,), jnp.float32) for _ in range(NBUF)],
          pltpu.VMEM((CHUNK,), jnp.float32),
          pltpu.VMEM((STAGE + LANES,), jnp.int32),
          [pltpu.VMEM((CHUNK,), jnp.int32) for _ in range(2)],
          pltpu.VMEM((SB * LANES,), jnp.int32),
          pltpu.VMEM_SHARED((F_pad,), jnp.float32),
          [pltpu.SemaphoreType.DMA for _ in range(NBUF)],
          [pltpu.SemaphoreType.DMA for _ in range(2)],
      ],
  )


def _tc_epilogue_kernel(F_pad, B):
  rows = F_pad // 128
  block_rows = next(b for b in (128, 112, 96, 80, 64, 56, 48, 40, 32, 24, 16, 8)
                    if rows % b == 0)
  grid = rows // block_rows

  def body(part_ref, bfm_ref, cv_ref, sat_ref, unsat_ref, acc_bv, acc_ms):
    g = pl.program_id(0)

    @pl.when(g == 0)
    def _():
      acc_bv[...] = jnp.zeros((B, 128), jnp.float32)
      acc_ms[...] = jnp.zeros((B, 128), jnp.float32)

    s = part_ref[0] + part_ref[1]
    cv = (s > 0.0).astype(jnp.float32)
    cv_ref[...] = cv
    b = bfm_ref[...]
    for k in range(B):
      m = b == k
      acc_bv[k:k + 1, :] += jnp.sum(jnp.where(m, cv, 0.0), axis=0,
                                    keepdims=True)
      acc_ms[k:k + 1, :] += jnp.sum(m.astype(jnp.float32), axis=0,
                                    keepdims=True)

    @pl.when(g == grid - 1)
    def _():
      bv = jnp.sum(acc_bv[...], axis=1, keepdims=True)
      ms = jnp.sum(acc_ms[...], axis=1, keepdims=True)
      sat_ref[...] = (ms == bv).astype(jnp.float32)
      unsat_ref[...] = ms - bv

  return pl.pallas_call(
      body,
      grid=(grid,),
      in_specs=[
          pl.BlockSpec((2, block_rows, 128), lambda g: (0, g, 0)),
          pl.BlockSpec((block_rows, 128), lambda g: (g, 0)),
      ],
      out_specs=[
          pl.BlockSpec((block_rows, 128), lambda g: (g, 0)),
          pl.BlockSpec((B, 1), lambda g: (0, 0)),
          pl.BlockSpec((B, 1), lambda g: (0, 0)),
      ],
      out_shape=[
          jax.ShapeDtypeStruct((rows, 128), jnp.float32),
          jax.ShapeDtypeStruct((B, 1), jnp.float32),
          jax.ShapeDtypeStruct((B, 1), jnp.float32),
      ],
      scratch_shapes=[
          pltpu.VMEM((B, 128), jnp.float32),
          pltpu.VMEM((B, 128), jnp.float32),
      ],
  )


@jax.jit
def kernel(variable_prediction, graph_map, batch_variable_map,
           batch_function_map, edge_feature):
  V = variable_prediction.shape[0]
  E = graph_map.shape[1]
  F = batch_function_map.shape[0]
  B = 16
  F_pad = ((F + 2047) // 2048) * 2048

  V_pad = ((V + 127) // 128) * 128
  vp = variable_prediction.reshape(V)
  ef1 = edge_feature.reshape(E)

  partial = _sc_edge_kernel(V, V_pad, E, F, F_pad)(vp, graph_map, ef1)

  bfm_pad = jnp.concatenate(
      [batch_function_map,
       jnp.full((F_pad - F,), B, jnp.int32)]).reshape(F_pad // 128, 128)
  part3 = partial.reshape(2, F_pad // 128, 128)

  cv, sat, unsat = _tc_epilogue_kernel(F_pad, B)(part3, bfm_pad)

  clause_values = cv.reshape(F_pad)[:F][:, None]
  return (sat, unsat, clause_values)
